# baseline (speedup 1.0000x reference)
import functools

import jax
import jax.numpy as jnp
from jax import lax
from jax.experimental import pallas as pl
from jax.experimental.pallas import tpu as pltpu
from jax.experimental.pallas import tpu_sc as plsc

NC = 2
NS = 16
CH = 128


def _lrelu(v):
    return jnp.where(v >= 0, v, v * 0.01)


def _mesh():
    return plsc.VectorSubcoreMesh(core_axis_name="c", subcore_axis_name="s",
                                  num_cores=NC, num_subcores=NS)


KB = 40


@functools.lru_cache(maxsize=None)
def _make_sc_deg(n_pad, kch):
    rows = n_pad // NS

    @functools.partial(
        pl.kernel,
        out_type=jax.ShapeDtypeStruct((NC, n_pad, 128), jnp.float32),
        mesh=_mesh(),
        scratch_types=[
            pltpu.VMEM((kch, CH), jnp.int32),
            pltpu.VMEM((CH, 128), jnp.float32),
            pltpu.VMEM_SHARED((n_pad, 128), jnp.float32),
            pltpu.SemaphoreType.DMA,
        ],
    )
    def sc_deg(dst_hbm, ones_hbm, zeros_hbm, out_hbm, dst_v, ones_v, deg_sh,
               sem):
        c = lax.axis_index("c")
        s = lax.axis_index("s")
        wid = c * NS + s
        r0 = s * rows
        pltpu.sync_copy(zeros_hbm, deg_sh.at[pl.ds(r0, rows)])
        pltpu.sync_copy(dst_hbm.at[pl.ds(wid * kch, kch)], dst_v)
        pltpu.sync_copy(ones_hbm, ones_v)
        plsc.subcore_barrier()

        def body(g, carry):
            for jj in range(8):
                pltpu.async_copy(ones_v, deg_sh.at[dst_v.at[g * 8 + jj]], sem,
                                 add=True)
            for jj in range(8):
                pltpu.make_async_copy(ones_v, deg_sh.at[dst_v.at[g * 8 + jj]],
                                      sem).wait()
            return carry

        lax.fori_loop(0, kch // 8, body, 0)
        plsc.subcore_barrier()
        pltpu.sync_copy(deg_sh.at[pl.ds(r0, rows)],
                        out_hbm.at[c, pl.ds(r0, rows)])

    return sc_deg


@functools.lru_cache(maxsize=None)
def _make_sc_agg(n_pad, kch0, kch1, d):
    rows = n_pad // NS

    @functools.partial(
        pl.kernel,
        out_type=jax.ShapeDtypeStruct((NC, n_pad, d), jnp.float32),
        mesh=_mesh(),
        scratch_types=[
            pltpu.VMEM((KB, CH), jnp.int32),
            pltpu.VMEM((KB, CH), jnp.int32),
            pltpu.VMEM((CH, d), jnp.float32),
            pltpu.VMEM((CH, d), jnp.float32),
            pltpu.VMEM_SHARED((n_pad, d), jnp.float32),
            pltpu.SemaphoreType.DMA,
            pltpu.SemaphoreType.DMA,
            pltpu.SemaphoreType.DMA,
            pltpu.SemaphoreType.DMA,
        ],
    )
    def sc_agg(xs_hbm, src_hbm, dst_hbm, zeros_hbm, out_hbm,
               src_v, dst_v, bufa, bufb, agg_sh, sema, semb, sca, scb):
        c = lax.axis_index("c")
        s = lax.axis_index("s")
        r0 = s * rows
        kc = jnp.where(c == 0, kch0, kch1)
        cbase = jnp.where(c == 0, s * kch0, NS * kch0 + s * kch1)
        pltpu.sync_copy(zeros_hbm, agg_sh.at[pl.ds(r0, rows)])
        plsc.subcore_barrier()

        def outer(b, carry):
            pltpu.sync_copy(src_hbm.at[pl.ds(cbase + b * KB, KB)], src_v)
            pltpu.sync_copy(dst_hbm.at[pl.ds(cbase + b * KB, KB)], dst_v)
            pltpu.async_copy(xs_hbm.at[src_v.at[0]], bufa, sema)
            pltpu.async_copy(xs_hbm.at[src_v.at[1]], bufb, semb)

            def body(jj, carry2):
                j0 = jj * 2
                j1 = j0 + 1
                pltpu.make_async_copy(xs_hbm.at[src_v.at[j0]], bufa, sema).wait()
                pltpu.async_copy(bufa, agg_sh.at[dst_v.at[j0]], sca, add=True)
                pltpu.make_async_copy(xs_hbm.at[src_v.at[j1]], bufb, semb).wait()
                pltpu.async_copy(bufb, agg_sh.at[dst_v.at[j1]], scb, add=True)
                pltpu.make_async_copy(bufa, agg_sh.at[dst_v.at[j0]], sca).wait()
                pltpu.async_copy(xs_hbm.at[src_v.at[j0 + 2]], bufa, sema)
                pltpu.make_async_copy(bufb, agg_sh.at[dst_v.at[j1]], scb).wait()
                pltpu.async_copy(xs_hbm.at[src_v.at[j1 + 2]], bufb, semb)
                return carry2

            lax.fori_loop(0, KB // 2 - 1, body, 0)
            pltpu.make_async_copy(xs_hbm.at[src_v.at[KB - 2]], bufa, sema).wait()
            pltpu.async_copy(bufa, agg_sh.at[dst_v.at[KB - 2]], sca, add=True)
            pltpu.make_async_copy(xs_hbm.at[src_v.at[KB - 1]], bufb, semb).wait()
            pltpu.async_copy(bufb, agg_sh.at[dst_v.at[KB - 1]], scb, add=True)
            pltpu.make_async_copy(bufa, agg_sh.at[dst_v.at[KB - 2]], sca).wait()
            pltpu.make_async_copy(bufb, agg_sh.at[dst_v.at[KB - 1]], scb).wait()
            return carry

        lax.fori_loop(0, kc // KB, outer, 0)
        plsc.subcore_barrier()
        pltpu.sync_copy(agg_sh.at[pl.ds(r0, rows)],
                        out_hbm.at[c, pl.ds(r0, rows)])

    return sc_agg


def _tc_norm(u_p, a_p, trans_w, trans_b2, n_users, n_real):
    n_pad, d = u_p.shape
    blk = 1024
    grid = (n_pad // blk,)

    def body(u_ref, a_ref, w_ref, b_ref, x_ref):
        i = pl.program_id(0)
        h = lax.dot_general(a_ref[...], w_ref[...], (((1,), (1,)), ((), ())),
                            preferred_element_type=jnp.float32)
        row = lax.broadcasted_iota(jnp.int32, (blk, 1), 0) + i * blk
        mask = (row >= n_users) & (row < n_real)
        xc = u_ref[...] + jnp.where(mask, h + b_ref[...], 0.0)
        nrm = jnp.sqrt(jnp.sum(xc * xc, axis=1, keepdims=True))
        x_ref[...] = xc / jnp.maximum(nrm, 1e-12)

    return pl.pallas_call(
        body,
        grid=grid,
        in_specs=[
            pl.BlockSpec((blk, d), lambda i: (i, 0)),
            pl.BlockSpec((blk, d), lambda i: (i, 0)),
            pl.BlockSpec((d, d), lambda i: (0, 0)),
            pl.BlockSpec((1, d), lambda i: (0, 0)),
        ],
        out_specs=pl.BlockSpec((blk, d), lambda i: (i, 0)),
        out_shape=jax.ShapeDtypeStruct((n_pad, d), jnp.float32),
    )(u_p, a_p, trans_w, trans_b2)


def _tc_scale(x, deg_parts):
    n_pad, d = x.shape
    blk = 1024
    grid = (n_pad // blk,)

    def body(x_ref, deg_ref, xs_ref, rd_ref):
        deg = jnp.sum(deg_ref[...], axis=(0, 2)) * (1.0 / 128.0)
        rdeg = lax.rsqrt(jnp.maximum(deg, 1.0))
        xs_ref[...] = x_ref[...] * rdeg[:, None]
        rd_ref[...] = jnp.broadcast_to(rdeg[:, None], (blk, d))

    return pl.pallas_call(
        body,
        grid=grid,
        in_specs=[
            pl.BlockSpec((blk, d), lambda i: (i, 0)),
            pl.BlockSpec((NC, blk, 128), lambda i: (0, i, 0)),
        ],
        out_specs=[
            pl.BlockSpec((blk, d), lambda i: (i, 0)),
            pl.BlockSpec((blk, d), lambda i: (i, 0)),
        ],
        out_shape=[
            jax.ShapeDtypeStruct((n_pad, d), jnp.float32),
            jax.ShapeDtypeStruct((n_pad, d), jnp.float32),
        ],
    )(x, deg_parts)


def _tc_layer(agg_parts, rdeg128, ne_p, w, dw, db2, ow, ob2, out_rows=None):
    _, n_pad, d = agg_parts.shape
    n_out = n_pad if out_rows is None else out_rows
    blk = 1024 if out_rows is None else out_rows // 10
    grid = (n_out // blk,)

    def body(ap_ref, rd_ref, ne_ref, w_ref, dw_ref, db_ref, ow_ref, ob_ref,
             xn_ref, xs_ref):
        rdeg = rd_ref[...]
        a = jnp.sum(ap_ref[...], axis=0) * rdeg
        hh = _lrelu(lax.dot_general(a, w_ref[...], (((1,), (0,)), ((), ())),
                                    preferred_element_type=jnp.float32))
        u = _lrelu(lax.dot_general(hh, dw_ref[...], (((1,), (1,)), ((), ())),
                                   preferred_element_type=jnp.float32)
                   + db_ref[...] + ne_ref[...])
        xn = _lrelu(lax.dot_general(hh, ow_ref[...], (((1,), (1,)), ((), ())),
                                    preferred_element_type=jnp.float32)
                    + ob_ref[...] + u)
        xn_ref[...] = xn
        xs_ref[...] = xn * rdeg

    return pl.pallas_call(
        body,
        grid=grid,
        in_specs=[
            pl.BlockSpec((NC, blk, d), lambda i: (0, i, 0)),
            pl.BlockSpec((blk, d), lambda i: (i, 0)),
            pl.BlockSpec((blk, d), lambda i: (i, 0)),
            pl.BlockSpec((d, d), lambda i: (0, 0)),
            pl.BlockSpec((d, d), lambda i: (0, 0)),
            pl.BlockSpec((1, d), lambda i: (0, 0)),
            pl.BlockSpec((d, d), lambda i: (0, 0)),
            pl.BlockSpec((1, d), lambda i: (0, 0)),
        ],
        out_specs=[
            pl.BlockSpec((blk, d), lambda i: (i, 0)),
            pl.BlockSpec((blk, d), lambda i: (i, 0)),
        ],
        out_shape=[
            jax.ShapeDtypeStruct((n_out, d), jnp.float32),
            jax.ShapeDtypeStruct((n_out, d), jnp.float32),
        ],
    )(agg_parts, rdeg128, ne_p, w, dw, db2, ow, ob2)


def kernel(feat, node_emb, edge_index, user_feat_emb, trans_w, trans_b,
           ws, des_w, des_b, outs_w, outs_b):
    n_users, d = user_feat_emb.shape
    n_items = feat.shape[0]
    n = n_users + n_items
    e = edge_index.shape[1]
    nw = NC * NS

    kt = -(-(2 * (-(-e // (nw * CH)))) // (2 * KB)) * (2 * KB)
    kch0 = min(max(KB, int(round(kt * 0.5 / KB)) * KB), kt - KB)
    kch1 = kt - kch0
    kch = kt // 2
    e_pad = NS * kt * CH
    n_pad = -(-(n + 128) // 2560) * 2560

    src = edge_index[0].astype(jnp.int32)
    dst = edge_index[1].astype(jnp.int32)
    pad_e = e_pad - e
    pad_idx = n + (jnp.arange(pad_e, dtype=jnp.int32) % 128)
    srcp = jnp.concatenate([src, pad_idx]).reshape(e_pad // CH, CH)
    dstp = jnp.concatenate([dst, pad_idx]).reshape(e_pad // CH, CH)

    rows = n_pad // NS
    zeros_big = jnp.zeros((rows, d), jnp.float32)
    ones128 = jnp.ones((CH, d), jnp.float32)

    u_p = jnp.pad(user_feat_emb, ((0, n_pad - n_users), (0, 0)))
    a_p = jnp.pad(feat, ((n_users, n_pad - n), (0, 0)))
    ne_p = jnp.pad(node_emb, ((0, n_pad - n), (0, 0)))

    deg_parts = _make_sc_deg(n_pad, kch)(dstp, ones128, zeros_big)
    x0 = _tc_norm(u_p, a_p, trans_w, trans_b.reshape(1, d), n_users, n)
    xs, rdeg128 = _tc_scale(x0, deg_parts)

    sc_agg = _make_sc_agg(n_pad, kch0, kch1, d)
    last_rows = n if (n % 10 == 0 and (n // 10) % 8 == 0) else None
    xn = None
    for i in range(len(ws)):
        agg_parts = sc_agg(xs, srcp, dstp, zeros_big)
        last = i == len(ws) - 1
        xn, xs = _tc_layer(agg_parts, rdeg128, ne_p, ws[i], des_w[i],
                           des_b[i].reshape(1, d), outs_w[i],
                           outs_b[i].reshape(1, d),
                           out_rows=last_rows if last else None)
    return (xn[:n], user_feat_emb)


# speedup vs baseline: 1.2117x; 1.2117x over previous
import functools

import jax
import jax.numpy as jnp
from jax import lax
from jax.experimental import pallas as pl
from jax.experimental.pallas import tpu as pltpu
from jax.experimental.pallas import tpu_sc as plsc

NC = 2
NS = 16
CH = 128


def _lrelu(v):
    return jnp.where(v >= 0, v, v * 0.01)


def _mesh():
    return plsc.VectorSubcoreMesh(core_axis_name="c", subcore_axis_name="s",
                                  num_cores=NC, num_subcores=NS)


KB = 40


@functools.lru_cache(maxsize=None)
def _make_sc_deg(n_pad, kch):
    rows = n_pad // NS

    @functools.partial(
        pl.kernel,
        out_type=jax.ShapeDtypeStruct((NC, n_pad, 128), jnp.float32),
        mesh=_mesh(),
        scratch_types=[
            pltpu.VMEM((kch, CH), jnp.int32),
            pltpu.VMEM((CH, 128), jnp.float32),
            pltpu.VMEM_SHARED((n_pad, 128), jnp.float32),
            pltpu.SemaphoreType.DMA,
        ],
    )
    def sc_deg(dst_hbm, ones_hbm, zeros_hbm, out_hbm, dst_v, ones_v, deg_sh,
               sem):
        c = lax.axis_index("c")
        s = lax.axis_index("s")
        wid = c * NS + s
        r0 = s * rows
        pltpu.sync_copy(zeros_hbm, deg_sh.at[pl.ds(r0, rows)])
        pltpu.sync_copy(dst_hbm.at[pl.ds(wid * kch, kch)], dst_v)
        pltpu.sync_copy(ones_hbm, ones_v)
        plsc.subcore_barrier()

        def body(g, carry):
            for jj in range(8):
                pltpu.async_copy(ones_v, deg_sh.at[dst_v.at[g * 8 + jj]], sem,
                                 add=True)
            for jj in range(8):
                pltpu.make_async_copy(ones_v, deg_sh.at[dst_v.at[g * 8 + jj]],
                                      sem).wait()
            return carry

        lax.fori_loop(0, kch // 8, body, 0)
        plsc.subcore_barrier()
        pltpu.sync_copy(deg_sh.at[pl.ds(r0, rows)],
                        out_hbm.at[c, pl.ds(r0, rows)])

    return sc_deg


@functools.lru_cache(maxsize=None)
def _make_sc_agg(n_pad, kch0, kch1, d):
    rows = n_pad // NS

    @functools.partial(
        pl.kernel,
        out_type=jax.ShapeDtypeStruct((NC, n_pad, d), jnp.float32),
        mesh=_mesh(),
        scratch_types=[
            pltpu.VMEM((KB, CH), jnp.int32),
            pltpu.VMEM((KB, CH), jnp.int32),
            pltpu.VMEM((CH, d), jnp.float32),
            pltpu.VMEM((CH, d), jnp.float32),
            pltpu.VMEM_SHARED((n_pad, d), jnp.float32),
            pltpu.SemaphoreType.DMA,
            pltpu.SemaphoreType.DMA,
        ],
    )
    def sc_agg(xs_hbm, src_hbm, dst_hbm, zeros_hbm, out_hbm,
               src_v, dst_v, bufa, bufb, agg_sh, sema, semb):
        c = lax.axis_index("c")
        s = lax.axis_index("s")
        r0 = s * rows
        kc = jnp.where(c == 0, kch0, kch1)
        cbase = jnp.where(c == 0, s * kch0, NS * kch0 + s * kch1)
        pltpu.sync_copy(zeros_hbm, agg_sh.at[pl.ds(r0, rows)])
        plsc.subcore_barrier()

        def outer(b, carry):
            pltpu.sync_copy(src_hbm.at[pl.ds(cbase + b * KB, KB)], src_v)
            pltpu.sync_copy(dst_hbm.at[pl.ds(cbase + b * KB, KB)], dst_v)
            pltpu.async_copy(xs_hbm.at[src_v.at[0]], bufa, sema)
            pltpu.async_copy(xs_hbm.at[src_v.at[1]], bufb, semb)

            def body(jj, carry2):
                j0 = jj * 2
                j1 = j0 + 1
                pltpu.make_async_copy(xs_hbm.at[src_v.at[j0]], bufa, sema).wait()
                pltpu.sync_copy(bufa, agg_sh.at[dst_v.at[j0]], add=True)
                pltpu.async_copy(xs_hbm.at[src_v.at[j0 + 2]], bufa, sema)
                pltpu.make_async_copy(xs_hbm.at[src_v.at[j1]], bufb, semb).wait()
                pltpu.sync_copy(bufb, agg_sh.at[dst_v.at[j1]], add=True)
                pltpu.async_copy(xs_hbm.at[src_v.at[j1 + 2]], bufb, semb)
                return carry2

            lax.fori_loop(0, KB // 2 - 1, body, 0)
            pltpu.make_async_copy(xs_hbm.at[src_v.at[KB - 2]], bufa, sema).wait()
            pltpu.sync_copy(bufa, agg_sh.at[dst_v.at[KB - 2]], add=True)
            pltpu.make_async_copy(xs_hbm.at[src_v.at[KB - 1]], bufb, semb).wait()
            pltpu.sync_copy(bufb, agg_sh.at[dst_v.at[KB - 1]], add=True)
            return carry

        lax.fori_loop(0, kc // KB, outer, 0)
        plsc.subcore_barrier()
        pltpu.sync_copy(agg_sh.at[pl.ds(r0, rows)],
                        out_hbm.at[c, pl.ds(r0, rows)])

    return sc_agg


def _tc_norm(u_p, a_p, trans_w, trans_b2, n_users, n_real):
    n_pad, d = u_p.shape
    blk = 1024
    grid = (n_pad // blk,)

    def body(u_ref, a_ref, w_ref, b_ref, x_ref):
        i = pl.program_id(0)
        h = lax.dot_general(a_ref[...], w_ref[...], (((1,), (1,)), ((), ())),
                            preferred_element_type=jnp.float32)
        row = lax.broadcasted_iota(jnp.int32, (blk, 1), 0) + i * blk
        mask = (row >= n_users) & (row < n_real)
        xc = u_ref[...] + jnp.where(mask, h + b_ref[...], 0.0)
        nrm = jnp.sqrt(jnp.sum(xc * xc, axis=1, keepdims=True))
        x_ref[...] = xc / jnp.maximum(nrm, 1e-12)

    return pl.pallas_call(
        body,
        grid=grid,
        in_specs=[
            pl.BlockSpec((blk, d), lambda i: (i, 0)),
            pl.BlockSpec((blk, d), lambda i: (i, 0)),
            pl.BlockSpec((d, d), lambda i: (0, 0)),
            pl.BlockSpec((1, d), lambda i: (0, 0)),
        ],
        out_specs=pl.BlockSpec((blk, d), lambda i: (i, 0)),
        out_shape=jax.ShapeDtypeStruct((n_pad, d), jnp.float32),
    )(u_p, a_p, trans_w, trans_b2)


def _tc_scale(x, deg_parts):
    n_pad, d = x.shape
    blk = 1024
    grid = (n_pad // blk,)

    def body(x_ref, deg_ref, xs_ref, rd_ref):
        deg = jnp.sum(deg_ref[...], axis=(0, 2)) * (1.0 / 128.0)
        rdeg = lax.rsqrt(jnp.maximum(deg, 1.0))
        xs_ref[...] = x_ref[...] * rdeg[:, None]
        rd_ref[...] = jnp.broadcast_to(rdeg[:, None], (blk, d))

    return pl.pallas_call(
        body,
        grid=grid,
        in_specs=[
            pl.BlockSpec((blk, d), lambda i: (i, 0)),
            pl.BlockSpec((NC, blk, 128), lambda i: (0, i, 0)),
        ],
        out_specs=[
            pl.BlockSpec((blk, d), lambda i: (i, 0)),
            pl.BlockSpec((blk, d), lambda i: (i, 0)),
        ],
        out_shape=[
            jax.ShapeDtypeStruct((n_pad, d), jnp.float32),
            jax.ShapeDtypeStruct((n_pad, d), jnp.float32),
        ],
    )(x, deg_parts)


def _tc_layer(agg_parts, rdeg128, ne_p, w, dw, db2, ow, ob2, out_rows=None):
    _, n_pad, d = agg_parts.shape
    n_out = n_pad if out_rows is None else out_rows
    blk = 1024 if out_rows is None else out_rows // 10
    grid = (n_out // blk,)

    def body(ap_ref, rd_ref, ne_ref, w_ref, dw_ref, db_ref, ow_ref, ob_ref,
             xn_ref, xs_ref):
        rdeg = rd_ref[...]
        a = jnp.sum(ap_ref[...], axis=0) * rdeg
        hh = _lrelu(lax.dot_general(a, w_ref[...], (((1,), (0,)), ((), ())),
                                    preferred_element_type=jnp.float32))
        u = _lrelu(lax.dot_general(hh, dw_ref[...], (((1,), (1,)), ((), ())),
                                   preferred_element_type=jnp.float32)
                   + db_ref[...] + ne_ref[...])
        xn = _lrelu(lax.dot_general(hh, ow_ref[...], (((1,), (1,)), ((), ())),
                                    preferred_element_type=jnp.float32)
                    + ob_ref[...] + u)
        xn_ref[...] = xn
        xs_ref[...] = xn * rdeg

    return pl.pallas_call(
        body,
        grid=grid,
        in_specs=[
            pl.BlockSpec((NC, blk, d), lambda i: (0, i, 0)),
            pl.BlockSpec((blk, d), lambda i: (i, 0)),
            pl.BlockSpec((blk, d), lambda i: (i, 0)),
            pl.BlockSpec((d, d), lambda i: (0, 0)),
            pl.BlockSpec((d, d), lambda i: (0, 0)),
            pl.BlockSpec((1, d), lambda i: (0, 0)),
            pl.BlockSpec((d, d), lambda i: (0, 0)),
            pl.BlockSpec((1, d), lambda i: (0, 0)),
        ],
        out_specs=[
            pl.BlockSpec((blk, d), lambda i: (i, 0)),
            pl.BlockSpec((blk, d), lambda i: (i, 0)),
        ],
        out_shape=[
            jax.ShapeDtypeStruct((n_out, d), jnp.float32),
            jax.ShapeDtypeStruct((n_out, d), jnp.float32),
        ],
    )(agg_parts, rdeg128, ne_p, w, dw, db2, ow, ob2)


def kernel(feat, node_emb, edge_index, user_feat_emb, trans_w, trans_b,
           ws, des_w, des_b, outs_w, outs_b):
    n_users, d = user_feat_emb.shape
    n_items = feat.shape[0]
    n = n_users + n_items
    e = edge_index.shape[1]
    nw = NC * NS

    kt = -(-(2 * (-(-e // (nw * CH)))) // (2 * KB)) * (2 * KB)
    kch0 = min(max(KB, int(round(kt * 0.5 / KB)) * KB), kt - KB)
    kch1 = kt - kch0
    kch = kt // 2
    e_pad = NS * kt * CH
    n_pad = -(-(n + 128) // 2560) * 2560

    src = edge_index[0].astype(jnp.int32)
    dst = edge_index[1].astype(jnp.int32)
    pad_e = e_pad - e
    pad_idx = n + (jnp.arange(pad_e, dtype=jnp.int32) % 128)
    srcp = jnp.concatenate([src, pad_idx]).reshape(e_pad // CH, CH)
    dstp = jnp.concatenate([dst, pad_idx]).reshape(e_pad // CH, CH)

    rows = n_pad // NS
    zeros_big = jnp.zeros((rows, d), jnp.float32)
    ones128 = jnp.ones((CH, d), jnp.float32)

    u_p = jnp.pad(user_feat_emb, ((0, n_pad - n_users), (0, 0)))
    a_p = jnp.pad(feat, ((n_users, n_pad - n), (0, 0)))
    ne_p = jnp.pad(node_emb, ((0, n_pad - n), (0, 0)))

    deg_parts = _make_sc_deg(n_pad, kch)(dstp, ones128, zeros_big)
    x0 = _tc_norm(u_p, a_p, trans_w, trans_b.reshape(1, d), n_users, n)
    xs, rdeg128 = _tc_scale(x0, deg_parts)

    sc_agg = _make_sc_agg(n_pad, kch0, kch1, d)
    last_rows = n if (n % 10 == 0 and (n // 10) % 8 == 0) else None
    xn = None
    for i in range(len(ws)):
        agg_parts = sc_agg(xs, srcp, dstp, zeros_big)
        last = i == len(ws) - 1
        xn, xs = _tc_layer(agg_parts, rdeg128, ne_p, ws[i], des_w[i],
                           des_b[i].reshape(1, d), outs_w[i],
                           outs_b[i].reshape(1, d),
                           out_rows=last_rows if last else None)
    return (xn[:n], user_feat_emb)
